# baseline (speedup 1.0000x reference)
import functools

import jax
import jax.numpy as jnp
from jax import lax
from jax.experimental import pallas as pl
from jax.experimental.pallas import tpu as pltpu
from jax.experimental.pallas import tpu_sc as plsc

NC = 2
NS = 16
L = 16
NW = NC * NS
D = 256
G = 16
DC = D // L
N = 50000

BT = 2048
NBT = 19
NTC = NBT * BT
CHUNK = 344
SZS = (120, 120, 104)
BR = 120
TAIL = N - NTC - NW * CHUNK

_mesh = plsc.VectorSubcoreMesh(core_axis_name="c", subcore_axis_name="s")


@functools.partial(
    pl.kernel,
    out_type=jax.ShapeDtypeStruct((NW, G), jnp.float32),
    mesh=_mesh,
    scratch_types=[
        pltpu.VMEM((CHUNK + L,), jnp.int32),
        pltpu.VMEM((TAIL + L,), jnp.int32),
        pltpu.VMEM((3, BR, D), jnp.float32),
        pltpu.VMEM((G, D), jnp.float32),
        pltpu.VMEM((D,), jnp.float32),
        pltpu.VMEM((G,), jnp.float32),
        pltpu.SemaphoreType.DMA,
        pltpu.SemaphoreType.DMA,
        pltpu.SemaphoreType.DMA,
    ],
)
def _sc_pool(nodes_h, idx_h, wp_h, out_h,
             idx_v, idx_t, buf_v, acc_v, wp_v, out_v,
             sem0, sem1, sem2):
    cid = lax.axis_index("c")
    sid = lax.axis_index("s")
    wid = cid * NS + sid
    base = NTC + wid * CHUNK
    sems = (sem0, sem1, sem2)
    offs = [sum(SZS[:k]) for k in range(len(SZS))]

    def _node_copy(blk, b):
        return pltpu.make_async_copy(
            nodes_h.at[pl.ds(base + offs[blk], SZS[blk])],
            buf_v.at[b, pl.ds(0, SZS[blk])],
            sems[b])

    _node_copy(0, 0).start()
    _node_copy(1, 1).start()
    pltpu.sync_copy(idx_h.at[pl.ds(base, CHUNK)], idx_v.at[pl.ds(0, CHUNK)])
    pltpu.sync_copy(wp_h, wp_v)

    zeros = jnp.zeros((L,), jnp.float32)

    def _zero(i, _):
        for c in range(DC):
            acc_v[i, pl.ds(c * L, L)] = zeros
        return 0

    lax.fori_loop(0, G, _zero, 0)

    zregs = tuple(jnp.zeros((L,), jnp.float32) for _ in range(DC))

    def _accum_rows(idx_ref, idx_off, n_rows, b):
        g_first = idx_ref[pl.ds(idx_off, L)][0]
        g_last = idx_ref[pl.ds(idx_off + n_rows - 1, L)][0]

        @pl.when(g_first == g_last)
        def _():
            def _row(i, regs):
                return tuple(regs[c] + buf_v[b, i, pl.ds(c * L, L)]
                             for c in range(DC))

            regs = lax.fori_loop(0, n_rows, _row, zregs, unroll=4)
            for c in range(DC):
                plsc.addupdate(acc_v.at[g_first, pl.ds(c * L, L)], regs[c])

        @pl.when(g_first != g_last)
        def _():
            def _row(i, _):
                g = idx_ref[pl.ds(idx_off + i, L)][0]
                for c in range(DC):
                    plsc.addupdate(acc_v.at[g, pl.ds(c * L, L)],
                                   buf_v[b, i, pl.ds(c * L, L)])
                return 0

            lax.fori_loop(0, n_rows, _row, 0)

    for blk in range(len(SZS)):
        b = blk % 3
        cp = _node_copy(blk, b)
        if blk + 2 < len(SZS):
            _node_copy(blk + 2, (blk + 2) % 3).start()
        cp.wait()
        _accum_rows(idx_v, offs[blk], SZS[blk], b)

    @pl.when(wid == NW - 1)
    def _():
        pltpu.sync_copy(idx_h.at[pl.ds(N - TAIL, TAIL)],
                        idx_t.at[pl.ds(0, TAIL)])
        pltpu.sync_copy(nodes_h.at[pl.ds(N - TAIL, TAIL)],
                        buf_v.at[0, pl.ds(0, TAIL)])
        _accum_rows(idx_t, 0, TAIL, 0)

    lane = lax.iota(jnp.int32, L)
    lv = zeros
    for g in range(G):
        def _c(c, pv, g=g):
            return pv + acc_v[g, pl.ds(c * L, L)] * wp_v[pl.ds(c * L, L)]

        pv = lax.fori_loop(0, DC, _c, jnp.zeros((L,), jnp.float32))
        s = pv[0]
        for j in range(1, L):
            s = s + pv[j]
        lv = jnp.where(lane == g, lv + s, lv)
    out_v[...] = lv
    pltpu.sync_copy(out_v, out_h.at[wid])


def _tc_body(idx_ref, x_ref, glob_ref, w_ref, b_ref, log_ref):
    step = pl.program_id(0)
    wp = w_ref[0:D, :]
    oh = (jnp.reshape(idx_ref[...], (1, BT)) ==
          lax.broadcasted_iota(jnp.int32, (G, BT), 0)).astype(jnp.float32)
    part = jnp.dot(oh, x_ref[...], preferred_element_type=jnp.float32)

    @pl.when(step == 0)
    def _():
        wg = w_ref[D:2 * D, :]
        log_ref[...] = (jnp.dot(glob_ref[...], wg,
                                preferred_element_type=jnp.float32) +
                        b_ref[0, 0])

    log_ref[...] += jnp.dot(part, wp, preferred_element_type=jnp.float32)


_tc_pool = pl.pallas_call(
    _tc_body,
    grid=(NBT,),
    in_specs=[
        pl.BlockSpec((1, 1, BT), lambda i: (i, 0, 0)),
        pl.BlockSpec((BT, D), lambda i: (i, 0)),
        pl.BlockSpec((G, D), lambda i: (0, 0)),
        pl.BlockSpec((2 * D, 1), lambda i: (0, 0)),
        pl.BlockSpec((1, 1), lambda i: (0, 0)),
    ],
    out_specs=pl.BlockSpec((G, 1), lambda i: (0, 0)),
    out_shape=jax.ShapeDtypeStruct((G, 1), jnp.float32),
)


def kernel(nodes, edges, senders, receivers, global_latent, node_graph_idx,
           W, b):
    idx = node_graph_idx.astype(jnp.int32)
    wp = W[:D, 0].astype(jnp.float32)
    idx_tc = idx[:NTC].reshape(NBT, 1, BT)
    log_tc = _tc_pool(idx_tc, nodes, global_latent,
                      W.astype(jnp.float32), b.reshape(1, 1))
    parts = _sc_pool(nodes, idx, wp)
    return log_tc + parts.sum(axis=0).reshape(G, 1)
